# baseline (speedup 1.0000x reference)
import functools

import jax
import jax.numpy as jnp
from jax import lax
from jax.experimental import pallas as pl
from jax.experimental.pallas import tpu as pltpu
from jax.experimental.pallas import tpu_sc as plsc

N = 4096 * 4096
J = N // 2
H1 = 32768
H2 = 32768
MASK31 = 0x7FFFFFFF

_info = plsc.get_sparse_core_info()
NC, NS, L = _info.num_cores, _info.num_subcores, _info.num_lanes
NW = NC * NS
PER_W = N // NW
ROWS_W = 4096 // NW
CROWS = 8
NCHUNK = ROWS_W // CROWS

_mesh = plsc.VectorSubcoreMesh(core_axis_name="c", subcore_axis_name="s")
_sc_params = pltpu.CompilerParams(needs_layout_passes=False)


def _zero_hist(hist_v, nbins):
    z = jnp.zeros((L,), jnp.int32)

    @plsc.parallel_loop(0, nbins // L, unroll=8)
    def _(i):
        hist_v[pl.ds(i * L, L)] = z


@functools.partial(
    pl.kernel,
    mesh=_mesh,
    out_type=jax.ShapeDtypeStruct((NW, H1), jnp.int32),
    compiler_params=_sc_params,
    scratch_types=[
        pltpu.VMEM((CROWS, 4096), jnp.float32),
        pltpu.VMEM((CROWS, 4096), jnp.float32),
        pltpu.VMEM((H1,), jnp.int32),
        pltpu.SemaphoreType.DMA,
        pltpu.SemaphoreType.DMA,
    ],
)
def _sc_hist1(x_hbm, hist_hbm, buf0, buf1, hist_v, sem0, sem1):
    wid = lax.axis_index("s") * NC + lax.axis_index("c")
    row0 = wid * ROWS_W
    bufs, sems = (buf0, buf1), (sem0, sem1)
    copies = [None, None]
    copies[0] = pltpu.async_copy(
        x_hbm.at[pl.ds(row0, CROWS), :], buf0, sem0)
    _zero_hist(hist_v, H1)
    ones = jnp.ones((L,), jnp.int32)

    for g in range(NCHUNK):
        b = g % 2
        if g + 1 < NCHUNK:
            nb = (g + 1) % 2
            copies[nb] = pltpu.async_copy(
                x_hbm.at[pl.ds(row0 + (g + 1) * CROWS, CROWS), :], bufs[nb],
                sems[nb])
        copies[b].wait()
        buf = bufs[b]

        @plsc.parallel_loop(0, CROWS * 4096 // L, unroll=16)
        def _(i):
            r = lax.shift_right_logical(i, 8)
            c = (i & jnp.int32(255)) * L
            v = buf[r, pl.ds(c, L)]
            k = lax.bitcast_convert_type(v, jnp.int32) & jnp.int32(MASK31)
            bn = lax.shift_right_logical(k, 16)
            plsc.addupdate_scatter(hist_v, [bn], ones)

    pltpu.sync_copy(hist_v, hist_hbm.at[wid])


@functools.partial(
    pl.kernel,
    mesh=_mesh,
    out_type=jax.ShapeDtypeStruct((NW, H2), jnp.int32),
    compiler_params=_sc_params,
    scratch_types=[
        pltpu.VMEM((CROWS, 4096), jnp.float32),
        pltpu.VMEM((CROWS, 4096), jnp.float32),
        pltpu.VMEM((H2,), jnp.int32),
        pltpu.VMEM((128,), jnp.int32),
        pltpu.SemaphoreType.DMA,
        pltpu.SemaphoreType.DMA,
    ],
)
def _sc_hist2(x_hbm, b_hbm, hist_hbm, buf0, buf1, hist_v, bbuf, sem0, sem1):
    wid = lax.axis_index("s") * NC + lax.axis_index("c")
    row0 = wid * ROWS_W
    bufs, sems = (buf0, buf1), (sem0, sem1)
    copies = [None, None]
    copies[0] = pltpu.async_copy(
        x_hbm.at[pl.ds(row0, CROWS), :], buf0, sem0)
    _zero_hist(hist_v, H2)
    pltpu.sync_copy(b_hbm, bbuf)
    bvec = bbuf[pl.ds(0, L)]
    ones = jnp.ones((L,), jnp.int32)

    for g in range(NCHUNK):
        b = g % 2
        if g + 1 < NCHUNK:
            nb = (g + 1) % 2
            copies[nb] = pltpu.async_copy(
                x_hbm.at[pl.ds(row0 + (g + 1) * CROWS, CROWS), :], bufs[nb],
                sems[nb])
        copies[b].wait()
        buf = bufs[b]

        @plsc.parallel_loop(0, CROWS * 4096 // L, unroll=16)
        def _(i):
            r = lax.shift_right_logical(i, 8)
            c = (i & jnp.int32(255)) * L
            v = buf[r, pl.ds(c, L)]
            k = lax.bitcast_convert_type(v, jnp.int32) & jnp.int32(MASK31)
            hi = lax.shift_right_logical(k, 16)
            lo = lax.shift_right_logical(k, 1) & jnp.int32(0x7FFF)
            plsc.addupdate_scatter(hist_v, [lo], ones, mask=hi == bvec)

    pltpu.sync_copy(hist_v, hist_hbm.at[wid])


def _cum_le(h):
    R = h.shape[0]
    i0 = lax.broadcasted_iota(jnp.int32, (128, 128), 0)
    i1 = lax.broadcasted_iota(jnp.int32, (128, 128), 1)
    within = (i0 <= i1).astype(jnp.float32)
    cumrow = jnp.dot(h, within, preferred_element_type=jnp.float32,
                     precision=lax.Precision.HIGHEST)
    r0 = lax.broadcasted_iota(jnp.int32, (R, R), 0)
    r1 = lax.broadcasted_iota(jnp.int32, (R, R), 1)
    strict = (r1 < r0).astype(jnp.float32)
    s = jnp.sum(h, axis=1, keepdims=True)
    prefix = jnp.dot(strict, s, preferred_element_type=jnp.float32,
                     precision=lax.Precision.HIGHEST)
    return cumrow + prefix


def _select1_body(h_ref, b128_ref, b11_ref, r11_ref):
    h = jnp.sum(h_ref[...].astype(jnp.float32), axis=0)
    cum = _cum_le(h)
    jf = jnp.float32(J)
    bf = jnp.sum((cum <= jf).astype(jnp.float32))
    b = bf.astype(jnp.int32)
    f0 = lax.broadcasted_iota(jnp.int32, (256, 128), 0)
    f1 = lax.broadcasted_iota(jnp.int32, (256, 128), 1)
    sel = (f0 * 128 + f1) == b
    c_b = jnp.sum(jnp.where(sel, h, 0.0))
    cum_b = jnp.sum(jnp.where(sel, cum, 0.0))
    r_rem = jf - (cum_b - c_b)
    b128_ref[...] = jnp.full((1, 128), b, jnp.int32)
    b11_ref[0, 0] = b
    r11_ref[0, 0] = r_rem.astype(jnp.int32)


_select1 = pl.pallas_call(
    _select1_body,
    out_shape=(
        jax.ShapeDtypeStruct((1, 128), jnp.int32),
        jax.ShapeDtypeStruct((1, 1), jnp.int32),
        jax.ShapeDtypeStruct((1, 1), jnp.int32),
    ),
    out_specs=(
        pl.BlockSpec(memory_space=pltpu.VMEM),
        pl.BlockSpec(memory_space=pltpu.SMEM),
        pl.BlockSpec(memory_space=pltpu.SMEM),
    ),
)


def _mask_body(b_ref, r_ref, h_ref, x_ref, o_ref, t_scr):
    @pl.when(pl.program_id(0) == 0)
    def _():
        h = jnp.sum(h_ref[...].astype(jnp.float32), axis=0)
        cum = _cum_le(h)
        rf = r_ref[0, 0].astype(jnp.float32)
        lf = jnp.sum((cum <= rf).astype(jnp.float32))
        low = lf.astype(jnp.int32)
        t_scr[0, 0] = b_ref[0, 0] * jnp.int32(65536) + low * jnp.int32(2)

    t = t_scr[0, 0]
    k = lax.bitcast_convert_type(x_ref[...], jnp.int32) & jnp.int32(MASK31)
    o_ref[...] = jnp.where(k >= t, jnp.float32(1.0), jnp.float32(0.0))


_ROWS_PER_BLK = 256
_mask_call = pl.pallas_call(
    _mask_body,
    grid=(4096 // _ROWS_PER_BLK,),
    in_specs=[
        pl.BlockSpec(memory_space=pltpu.SMEM),
        pl.BlockSpec(memory_space=pltpu.SMEM),
        pl.BlockSpec((NW, 256, 128), lambda i: (0, 0, 0)),
        pl.BlockSpec((_ROWS_PER_BLK, 4096), lambda i: (i, 0)),
    ],
    out_specs=pl.BlockSpec((_ROWS_PER_BLK, 4096), lambda i: (i, 0)),
    out_shape=jax.ShapeDtypeStruct((4096, 4096), jnp.float32),
    scratch_shapes=[pltpu.SMEM((1, 1), jnp.int32)],
)


def kernel(param):
    hist1 = _sc_hist1(param)
    b128, b11, r11 = _select1(hist1.reshape(NW, 256, 128))
    hist2 = _sc_hist2(param, b128.reshape(128))
    return _mask_call(b11, r11, hist2.reshape(NW, 256, 128), param)


# speedup vs baseline: 1.0094x; 1.0094x over previous
import functools

import jax
import jax.numpy as jnp
from jax import lax
from jax.experimental import pallas as pl
from jax.experimental.pallas import tpu as pltpu
from jax.experimental.pallas import tpu_sc as plsc

N = 4096 * 4096
J = N // 2
H1 = 32768
H2 = 32768
MASK31 = 0x7FFFFFFF

_info = plsc.get_sparse_core_info()
NC, NS, L = _info.num_cores, _info.num_subcores, _info.num_lanes
NW = NC * NS
PER_W = N // NW
ROWS_W = 4096 // NW
CROWS = 8
NCHUNK = ROWS_W // CROWS

_mesh = plsc.VectorSubcoreMesh(core_axis_name="c", subcore_axis_name="s")
_sc_params = pltpu.CompilerParams(needs_layout_passes=False)


def _zero_hist(hist_v, nbins):
    z = jnp.zeros((L,), jnp.int32)

    @plsc.parallel_loop(0, nbins // L, unroll=8)
    def _(i):
        hist_v[pl.ds(i * L, L)] = z


@functools.partial(
    pl.kernel,
    mesh=_mesh,
    out_type=jax.ShapeDtypeStruct((NW, H1), jnp.int32),
    compiler_params=_sc_params,
    scratch_types=[
        pltpu.VMEM((CROWS, 4096), jnp.float32),
        pltpu.VMEM((CROWS, 4096), jnp.float32),
        pltpu.VMEM((H1,), jnp.int32),
        pltpu.SemaphoreType.DMA,
        pltpu.SemaphoreType.DMA,
    ],
)
def _sc_hist1(x_hbm, hist_hbm, buf0, buf1, hist_v, sem0, sem1):
    wid = lax.axis_index("s") * NC + lax.axis_index("c")
    row0 = wid * ROWS_W
    bufs, sems = (buf0, buf1), (sem0, sem1)
    copies = [None, None]
    copies[0] = pltpu.async_copy(
        x_hbm.at[pl.ds(row0, CROWS), :], buf0, sem0)
    _zero_hist(hist_v, H1)
    ones = jnp.ones((L,), jnp.int32)

    for g in range(NCHUNK):
        b = g % 2
        if g + 1 < NCHUNK:
            nb = (g + 1) % 2
            copies[nb] = pltpu.async_copy(
                x_hbm.at[pl.ds(row0 + (g + 1) * CROWS, CROWS), :], bufs[nb],
                sems[nb])
        copies[b].wait()
        buf = bufs[b]

        @plsc.parallel_loop(0, CROWS * 4096 // L, unroll=8)
        def _(i):
            r = lax.shift_right_logical(i, 8)
            c = (i & jnp.int32(255)) * L
            v = buf[r, pl.ds(c, L)]
            k = lax.bitcast_convert_type(v, jnp.int32) & jnp.int32(MASK31)
            bn = lax.shift_right_logical(k, 16)
            plsc.addupdate_scatter(hist_v, [bn], ones)

    pltpu.sync_copy(hist_v, hist_hbm.at[wid])


@functools.partial(
    pl.kernel,
    mesh=_mesh,
    out_type=jax.ShapeDtypeStruct((NW, H2), jnp.int32),
    compiler_params=_sc_params,
    scratch_types=[
        pltpu.VMEM((CROWS, 4096), jnp.float32),
        pltpu.VMEM((CROWS, 4096), jnp.float32),
        pltpu.VMEM((H2,), jnp.int32),
        pltpu.VMEM((128,), jnp.int32),
        pltpu.SemaphoreType.DMA,
        pltpu.SemaphoreType.DMA,
    ],
)
def _sc_hist2(x_hbm, b_hbm, hist_hbm, buf0, buf1, hist_v, bbuf, sem0, sem1):
    wid = lax.axis_index("s") * NC + lax.axis_index("c")
    row0 = wid * ROWS_W
    bufs, sems = (buf0, buf1), (sem0, sem1)
    copies = [None, None]
    copies[0] = pltpu.async_copy(
        x_hbm.at[pl.ds(row0, CROWS), :], buf0, sem0)
    _zero_hist(hist_v, H2)
    pltpu.sync_copy(b_hbm, bbuf)
    bvec = bbuf[pl.ds(0, L)]
    ones = jnp.ones((L,), jnp.int32)

    for g in range(NCHUNK):
        b = g % 2
        if g + 1 < NCHUNK:
            nb = (g + 1) % 2
            copies[nb] = pltpu.async_copy(
                x_hbm.at[pl.ds(row0 + (g + 1) * CROWS, CROWS), :], bufs[nb],
                sems[nb])
        copies[b].wait()
        buf = bufs[b]

        @plsc.parallel_loop(0, CROWS * 4096 // L, unroll=8)
        def _(i):
            r = lax.shift_right_logical(i, 8)
            c = (i & jnp.int32(255)) * L
            v = buf[r, pl.ds(c, L)]
            k = lax.bitcast_convert_type(v, jnp.int32) & jnp.int32(MASK31)
            hi = lax.shift_right_logical(k, 16)
            lo = lax.shift_right_logical(k, 1) & jnp.int32(0x7FFF)
            plsc.addupdate_scatter(hist_v, [lo], ones, mask=hi == bvec)

    pltpu.sync_copy(hist_v, hist_hbm.at[wid])


def _cum_le(h):
    R = h.shape[0]
    i0 = lax.broadcasted_iota(jnp.int32, (128, 128), 0)
    i1 = lax.broadcasted_iota(jnp.int32, (128, 128), 1)
    within = (i0 <= i1).astype(jnp.float32)
    cumrow = jnp.dot(h, within, preferred_element_type=jnp.float32,
                     precision=lax.Precision.HIGHEST)
    r0 = lax.broadcasted_iota(jnp.int32, (R, R), 0)
    r1 = lax.broadcasted_iota(jnp.int32, (R, R), 1)
    strict = (r1 < r0).astype(jnp.float32)
    s = jnp.sum(h, axis=1, keepdims=True)
    prefix = jnp.dot(strict, s, preferred_element_type=jnp.float32,
                     precision=lax.Precision.HIGHEST)
    return cumrow + prefix


def _select1_body(h_ref, b128_ref, b11_ref, r11_ref):
    h = jnp.sum(h_ref[...].astype(jnp.float32), axis=0)
    cum = _cum_le(h)
    jf = jnp.float32(J)
    bf = jnp.sum((cum <= jf).astype(jnp.float32))
    b = bf.astype(jnp.int32)
    f0 = lax.broadcasted_iota(jnp.int32, (256, 128), 0)
    f1 = lax.broadcasted_iota(jnp.int32, (256, 128), 1)
    sel = (f0 * 128 + f1) == b
    c_b = jnp.sum(jnp.where(sel, h, 0.0))
    cum_b = jnp.sum(jnp.where(sel, cum, 0.0))
    r_rem = jf - (cum_b - c_b)
    b128_ref[...] = jnp.full((1, 128), b, jnp.int32)
    b11_ref[0, 0] = b
    r11_ref[0, 0] = r_rem.astype(jnp.int32)


_select1 = pl.pallas_call(
    _select1_body,
    out_shape=(
        jax.ShapeDtypeStruct((1, 128), jnp.int32),
        jax.ShapeDtypeStruct((1, 1), jnp.int32),
        jax.ShapeDtypeStruct((1, 1), jnp.int32),
    ),
    out_specs=(
        pl.BlockSpec(memory_space=pltpu.VMEM),
        pl.BlockSpec(memory_space=pltpu.SMEM),
        pl.BlockSpec(memory_space=pltpu.SMEM),
    ),
)


def _mask_body(b_ref, r_ref, h_ref, x_ref, o_ref, t_scr):
    @pl.when(pl.program_id(0) == 0)
    def _():
        h = jnp.sum(h_ref[...].astype(jnp.float32), axis=0)
        cum = _cum_le(h)
        rf = r_ref[0, 0].astype(jnp.float32)
        lf = jnp.sum((cum <= rf).astype(jnp.float32))
        low = lf.astype(jnp.int32)
        t_scr[0, 0] = b_ref[0, 0] * jnp.int32(65536) + low * jnp.int32(2)

    t = t_scr[0, 0]
    k = lax.bitcast_convert_type(x_ref[...], jnp.int32) & jnp.int32(MASK31)
    o_ref[...] = jnp.where(k >= t, jnp.float32(1.0), jnp.float32(0.0))


_ROWS_PER_BLK = 256
_mask_call = pl.pallas_call(
    _mask_body,
    grid=(4096 // _ROWS_PER_BLK,),
    in_specs=[
        pl.BlockSpec(memory_space=pltpu.SMEM),
        pl.BlockSpec(memory_space=pltpu.SMEM),
        pl.BlockSpec((NW, 256, 128), lambda i: (0, 0, 0)),
        pl.BlockSpec((_ROWS_PER_BLK, 4096), lambda i: (i, 0)),
    ],
    out_specs=pl.BlockSpec((_ROWS_PER_BLK, 4096), lambda i: (i, 0)),
    out_shape=jax.ShapeDtypeStruct((4096, 4096), jnp.float32),
    scratch_shapes=[pltpu.SMEM((1, 1), jnp.int32)],
)


def kernel(param):
    hist1 = _sc_hist1(param)
    b128, b11, r11 = _select1(hist1.reshape(NW, 256, 128))
    hist2 = _sc_hist2(param, b128.reshape(128))
    return _mask_call(b11, r11, hist2.reshape(NW, 256, 128), param)


# speedup vs baseline: 1.0191x; 1.0096x over previous
import functools

import jax
import jax.numpy as jnp
from jax import lax
from jax.experimental import pallas as pl
from jax.experimental.pallas import tpu as pltpu
from jax.experimental.pallas import tpu_sc as plsc

N = 4096 * 4096
J = N // 2
H1 = 32768
H2 = 32768
MASK31 = 0x7FFFFFFF

_info = plsc.get_sparse_core_info()
NC, NS, L = _info.num_cores, _info.num_subcores, _info.num_lanes
NW = NC * NS
PER_W = N // NW
ROWS_W = 4096 // NW
CROWS = 8
NCHUNK = ROWS_W // CROWS

_mesh = plsc.VectorSubcoreMesh(core_axis_name="c", subcore_axis_name="s")
_sc_params = pltpu.CompilerParams(needs_layout_passes=False)


def _zero_hist(hist_v, nbins):
    z = jnp.zeros((L,), jnp.int32)

    @plsc.parallel_loop(0, nbins // L, unroll=8)
    def _(i):
        hist_v[pl.ds(i * L, L)] = z


@functools.partial(
    pl.kernel,
    mesh=_mesh,
    out_type=jax.ShapeDtypeStruct((NW, H1), jnp.int32),
    compiler_params=_sc_params,
    scratch_types=[
        pltpu.VMEM((CROWS, 4096), jnp.float32),
        pltpu.VMEM((CROWS, 4096), jnp.float32),
        pltpu.VMEM((H1,), jnp.int32),
        pltpu.SemaphoreType.DMA,
        pltpu.SemaphoreType.DMA,
    ],
)
def _sc_hist1(x_hbm, hist_hbm, buf0, buf1, hist_v, sem0, sem1):
    wid = lax.axis_index("s") * NC + lax.axis_index("c")
    row0 = wid * ROWS_W
    bufs, sems = (buf0, buf1), (sem0, sem1)
    copies = [None, None]
    copies[0] = pltpu.async_copy(
        x_hbm.at[pl.ds(row0, CROWS), :], buf0, sem0)
    _zero_hist(hist_v, H1)
    ones = jnp.ones((L,), jnp.int32)

    for g in range(NCHUNK):
        b = g % 2
        if g + 1 < NCHUNK:
            nb = (g + 1) % 2
            copies[nb] = pltpu.async_copy(
                x_hbm.at[pl.ds(row0 + (g + 1) * CROWS, CROWS), :], bufs[nb],
                sems[nb])
        copies[b].wait()
        buf = bufs[b]

        @plsc.parallel_loop(0, CROWS * 4096 // L, unroll=8)
        def _(i):
            r = lax.shift_right_logical(i, 8)
            c = (i & jnp.int32(255)) * L
            v = buf[r, pl.ds(c, L)]
            k = lax.bitcast_convert_type(v, jnp.int32) & jnp.int32(MASK31)
            bn = lax.shift_right_logical(k, 16)
            plsc.addupdate_scatter(hist_v, [bn], ones)

    pltpu.sync_copy(hist_v, hist_hbm.at[wid])


@functools.partial(
    pl.kernel,
    mesh=_mesh,
    out_type=jax.ShapeDtypeStruct((NW, H2), jnp.int32),
    compiler_params=_sc_params,
    scratch_types=[
        pltpu.VMEM((CROWS, 4096), jnp.float32),
        pltpu.VMEM((CROWS, 4096), jnp.float32),
        pltpu.VMEM((H2,), jnp.int32),
        pltpu.VMEM((128,), jnp.int32),
        pltpu.SemaphoreType.DMA,
        pltpu.SemaphoreType.DMA,
    ],
)
def _sc_hist2(x_hbm, b_hbm, hist_hbm, buf0, buf1, hist_v, bbuf, sem0, sem1):
    wid = lax.axis_index("s") * NC + lax.axis_index("c")
    row0 = wid * ROWS_W
    bufs, sems = (buf0, buf1), (sem0, sem1)
    copies = [None, None]
    copies[0] = pltpu.async_copy(
        x_hbm.at[pl.ds(row0, CROWS), :], buf0, sem0)
    _zero_hist(hist_v, H2)
    pltpu.sync_copy(b_hbm, bbuf)
    bvec = bbuf[pl.ds(0, L)]
    ones = jnp.ones((L,), jnp.int32)

    for g in range(NCHUNK):
        b = g % 2
        if g + 1 < NCHUNK:
            nb = (g + 1) % 2
            copies[nb] = pltpu.async_copy(
                x_hbm.at[pl.ds(row0 + (g + 1) * CROWS, CROWS), :], bufs[nb],
                sems[nb])
        copies[b].wait()
        buf = bufs[b]

        @plsc.parallel_loop(0, CROWS * 4096 // L, unroll=8)
        def _(i):
            r = lax.shift_right_logical(i, 8)
            c = (i & jnp.int32(255)) * L
            v = buf[r, pl.ds(c, L)]
            k = lax.bitcast_convert_type(v, jnp.int32) & jnp.int32(MASK31)
            hi = lax.shift_right_logical(k, 16)
            lo = lax.shift_right_logical(k, 1) & jnp.int32(0x7FFF)
            plsc.addupdate_scatter(hist_v, [lo], ones, mask=hi == bvec)

    pltpu.sync_copy(hist_v, hist_hbm.at[wid])


def _cum_le(h):
    R = h.shape[0]
    i0 = lax.broadcasted_iota(jnp.int32, (128, 128), 0)
    i1 = lax.broadcasted_iota(jnp.int32, (128, 128), 1)
    within = (i0 <= i1).astype(jnp.float32)
    cumrow = jnp.dot(h, within, preferred_element_type=jnp.float32,
                     precision=lax.Precision.HIGHEST)
    r0 = lax.broadcasted_iota(jnp.int32, (R, R), 0)
    r1 = lax.broadcasted_iota(jnp.int32, (R, R), 1)
    strict = (r1 < r0).astype(jnp.float32)
    s = jnp.sum(h, axis=1, keepdims=True)
    prefix = jnp.dot(strict, s, preferred_element_type=jnp.float32,
                     precision=lax.Precision.HIGHEST)
    return cumrow + prefix


def _select1_body(h_ref, b128_ref, b11_ref, r11_ref):
    h = jnp.sum(h_ref[...].astype(jnp.float32), axis=0)
    cum = _cum_le(h)
    jf = jnp.float32(J)
    bf = jnp.sum((cum <= jf).astype(jnp.float32))
    b = bf.astype(jnp.int32)
    f0 = lax.broadcasted_iota(jnp.int32, (256, 128), 0)
    f1 = lax.broadcasted_iota(jnp.int32, (256, 128), 1)
    sel = (f0 * 128 + f1) == b
    c_b = jnp.sum(jnp.where(sel, h, 0.0))
    cum_b = jnp.sum(jnp.where(sel, cum, 0.0))
    r_rem = jf - (cum_b - c_b)
    b128_ref[...] = jnp.full((1, 128), b, jnp.int32)
    b11_ref[0, 0] = b
    r11_ref[0, 0] = r_rem.astype(jnp.int32)


_select1 = pl.pallas_call(
    _select1_body,
    out_shape=(
        jax.ShapeDtypeStruct((1, 128), jnp.int32),
        jax.ShapeDtypeStruct((1, 1), jnp.int32),
        jax.ShapeDtypeStruct((1, 1), jnp.int32),
    ),
    out_specs=(
        pl.BlockSpec(memory_space=pltpu.VMEM),
        pl.BlockSpec(memory_space=pltpu.SMEM),
        pl.BlockSpec(memory_space=pltpu.SMEM),
    ),
)


def _mask_body(b_ref, r_ref, h_ref, x_ref, o_ref, t_scr):
    @pl.when(pl.program_id(0) == 0)
    def _():
        h = jnp.sum(h_ref[...].astype(jnp.float32), axis=0)
        cum = _cum_le(h)
        rf = r_ref[0, 0].astype(jnp.float32)
        lf = jnp.sum((cum <= rf).astype(jnp.float32))
        low = lf.astype(jnp.int32)
        t_scr[0, 0] = b_ref[0, 0] * jnp.int32(65536) + low * jnp.int32(2)

    t = t_scr[0, 0]
    k = lax.bitcast_convert_type(x_ref[...], jnp.int32) & jnp.int32(MASK31)
    o_ref[...] = jnp.where(k >= t, jnp.float32(1.0), jnp.float32(0.0))


_ROWS_PER_BLK = 512
_mask_call = pl.pallas_call(
    _mask_body,
    grid=(4096 // _ROWS_PER_BLK,),
    in_specs=[
        pl.BlockSpec(memory_space=pltpu.SMEM),
        pl.BlockSpec(memory_space=pltpu.SMEM),
        pl.BlockSpec((NW, 256, 128), lambda i: (0, 0, 0)),
        pl.BlockSpec((_ROWS_PER_BLK, 4096), lambda i: (i, 0)),
    ],
    out_specs=pl.BlockSpec((_ROWS_PER_BLK, 4096), lambda i: (i, 0)),
    out_shape=jax.ShapeDtypeStruct((4096, 4096), jnp.float32),
    scratch_shapes=[pltpu.SMEM((1, 1), jnp.int32)],
)


def kernel(param):
    hist1 = _sc_hist1(param)
    b128, b11, r11 = _select1(hist1.reshape(NW, 256, 128))
    hist2 = _sc_hist2(param, b128.reshape(128))
    return _mask_call(b11, r11, hist2.reshape(NW, 256, 128), param)
